# baseline (speedup 1.0000x reference)
import functools

import jax
import jax.numpy as jnp
import numpy as np
from jax import lax
from jax.experimental import pallas as pl
from jax.experimental.pallas import tpu as pltpu
from jax.experimental.pallas import tpu_sc as plsc

N_GRAPHS = 16
PI = float(np.pi)

_NC, _NS = 2, 16
_NW = _NC * _NS


def _sc_gather(table, idx):
    E = idx.shape[0]
    n, D = table.shape
    dt = table.dtype
    assert E % _NW == 0
    per_w = E // _NW
    C = 128
    nch = per_w // C
    tail = per_w - nch * C
    assert tail % 8 == 0
    NB = 4
    mesh = plsc.VectorSubcoreMesh(core_axis_name="c", subcore_axis_name="s",
                                  num_cores=_NC, num_subcores=_NS)

    @functools.partial(
        pl.kernel,
        out_type=jax.ShapeDtypeStruct((E, D), dt),
        mesh=mesh,
        scratch_types=[
            pltpu.VMEM((per_w,), jnp.int32),
            [pltpu.VMEM((C, D), dt) for _ in range(NB)],
            [pltpu.SemaphoreType.DMA for _ in range(NB)],
            [pltpu.SemaphoreType.DMA for _ in range(NB)],
            pltpu.VMEM((tail, D), dt) if tail else None,
            pltpu.SemaphoreType.DMA,
        ],
    )
    def k(tab_hbm, idx_hbm, out_hbm, idx_all, bufs, gsems, wsems, buf_t, sem_t):
        c = lax.axis_index("c")
        s = lax.axis_index("s")
        wid = s * _NC + c
        base = wid * per_w
        pltpu.sync_copy(idx_hbm.at[pl.ds(base, per_w)], idx_all)

        def start_g(ch, b):
            pltpu.async_copy(tab_hbm.at[idx_all.at[pl.ds(ch * C, C)]],
                             bufs[b], gsems[b])

        for b in range(NB):
            if b < nch:
                start_g(b, b)

        def outer(j0, _):
            for b in range(NB):
                ch = j0 + b

                @pl.when(ch < nch)
                def _():
                    pltpu.make_async_copy(tab_hbm.at[idx_all.at[pl.ds(ch * C, C)]],
                                          bufs[b], gsems[b]).wait()
                    w = pltpu.async_copy(bufs[b],
                                         out_hbm.at[pl.ds(base + ch * C, C)],
                                         wsems[b])

                    @pl.when(ch + NB < nch)
                    def _():
                        w.wait()
                        start_g(ch + NB, b)
            return 0

        nouter = -(-nch // NB)
        lax.fori_loop(0, nouter, lambda j, x: outer(j * NB, x), 0)
        for b in range(min(NB, nch)):
            pltpu.make_async_copy(bufs[b], out_hbm.at[pl.ds(base, C)],
                                  wsems[b]).wait()
        if tail:
            bt = base + nch * C
            pltpu.async_copy(tab_hbm.at[idx_all.at[pl.ds(nch * C, tail)]],
                             buf_t, sem_t).wait()
            pltpu.sync_copy(buf_t, out_hbm.at[pl.ds(bt, tail)])

    return k(table, idx)


def _sc_agg(hp, idx_g, idx_sc, n_out):
    E = idx_g.shape[0]
    n, D = hp.shape
    C = 128
    assert D == 128 and E % C == 0
    ntot = E // C
    nbase = ntot // _NW
    nrem = ntot - nbase * _NW
    rows_per_sub = -(-(n_out + 8) // (_NS * 8)) * 8
    npad = rows_per_sub * _NS
    z_d = jnp.zeros((rows_per_sub, D), jnp.float32)
    NB = 2
    mesh = plsc.VectorSubcoreMesh(core_axis_name="c", subcore_axis_name="s",
                                  num_cores=_NC, num_subcores=_NS)

    @functools.partial(
        pl.kernel,
        out_type=jax.ShapeDtypeStruct((_NC, npad, D), jnp.float32),
        mesh=mesh,
        scratch_types=[
            pltpu.VMEM(((nbase + 1) * C,), jnp.int32),
            [pltpu.VMEM((C,), jnp.int32) for _ in range(NB)],
            [pltpu.VMEM((C, D), jnp.float32) for _ in range(NB)],
            [pltpu.SemaphoreType.DMA for _ in range(NB)],
            [pltpu.SemaphoreType.DMA for _ in range(NB)],
            [pltpu.SemaphoreType.DMA for _ in range(NB)],
            pltpu.VMEM_SHARED((npad, D), jnp.float32),
        ],
    )
    def k(h_hbm, ig_hbm, is_hbm, zd_hbm, acc_out,
          ig_all, is_v, rows_v, semg, semi, sems, acc_sh):
        c = lax.axis_index("c")
        s = lax.axis_index("s")
        wid = s * _NC + c
        nch = nbase + (wid < nrem).astype(jnp.int32)
        base = (wid * nbase + jnp.minimum(wid, nrem)) * C
        row0 = s * rows_per_sub
        pltpu.sync_copy(zd_hbm, acc_sh.at[pl.ds(row0, rows_per_sub)])
        if nbase:
            pltpu.sync_copy(ig_hbm.at[pl.ds(base, nbase * C)],
                            ig_all.at[pl.ds(0, nbase * C)])

        @pl.when(wid < nrem)
        def _():
            pltpu.sync_copy(ig_hbm.at[pl.ds(base + nbase * C, C)],
                            ig_all.at[pl.ds(nbase * C, C)])

        plsc.subcore_barrier()

        def start_load(j, b):
            pltpu.async_copy(is_hbm.at[pl.ds(base + j * C, C)], is_v[b],
                             semi[b])
            pltpu.async_copy(h_hbm.at[ig_all.at[pl.ds(j * C, C)]], rows_v[b],
                             semg[b])

        for b in range(NB):
            @pl.when(b < nch)
            def _():
                start_load(b, b)

        def outer(j0, _):
            for b in range(NB):
                ch = j0 + b

                @pl.when(ch < nch)
                def _():
                    pltpu.make_async_copy(is_hbm.at[pl.ds(base, C)],
                                          is_v[b], semi[b]).wait()
                    pltpu.make_async_copy(h_hbm.at[ig_all.at[pl.ds(0, C)]],
                                          rows_v[b], semg[b]).wait()
                    w = pltpu.async_copy(rows_v[b], acc_sh.at[is_v[b]],
                                         sems[b], add=True)

                    @pl.when(ch + NB < nch)
                    def _():
                        w.wait()
                        start_load(ch + NB, b)

            return 0

        nouter = -(-(nbase + 1) // NB)
        lax.fori_loop(0, nouter, lambda j, x: outer(j * NB, x), 0)
        for b in range(NB):
            @pl.when(b < nch)
            def _():
                pltpu.make_async_copy(rows_v[b], acc_sh.at[is_v[b]],
                                      sems[b]).wait()
        plsc.subcore_barrier()
        pltpu.sync_copy(acc_sh.at[pl.ds(row0, rows_per_sub)],
                        acc_out.at[c, pl.ds(row0, rows_per_sub)])

    return k(hp, idx_g, idx_sc, z_d)


def _sc_gather2(tab_a, idx_a, tab_b, idx_b):
    E = idx_a.shape[0]
    D = tab_a.shape[1]
    dt = tab_a.dtype
    C = 128
    assert E % C == 0
    ntot = E // C
    nbase = ntot // _NW
    nrem = ntot - nbase * _NW
    NB = 3
    mesh = plsc.VectorSubcoreMesh(core_axis_name="c", subcore_axis_name="s",
                                  num_cores=_NC, num_subcores=_NS)

    @functools.partial(
        pl.kernel,
        out_type=[jax.ShapeDtypeStruct((E, D), dt),
                  jax.ShapeDtypeStruct((E, D), dt)],
        mesh=mesh,
        scratch_types=[
            [pltpu.VMEM(((nbase + 1) * C,), jnp.int32) for _ in range(2)],
            [pltpu.VMEM((C, D), dt) for _ in range(2 * NB)],
            [pltpu.SemaphoreType.DMA for _ in range(2 * NB)],
            [pltpu.SemaphoreType.DMA for _ in range(2 * NB)],
        ],
    )
    def k(ta_hbm, ia_hbm, tb_hbm, ib_hbm, oa_hbm, ob_hbm,
          idx_all, bufs, gsems, wsems):
        c = lax.axis_index("c")
        s = lax.axis_index("s")
        wid = s * _NC + c
        nch = nbase + (wid < nrem).astype(jnp.int32)
        base = (wid * nbase + jnp.minimum(wid, nrem)) * C
        for t, ihbm in enumerate((ia_hbm, ib_hbm)):
            if nbase:
                pltpu.sync_copy(ihbm.at[pl.ds(base, nbase * C)],
                                idx_all[t].at[pl.ds(0, nbase * C)])

            @pl.when(wid < nrem)
            def _():
                pltpu.sync_copy(ihbm.at[pl.ds(base + nbase * C, C)],
                                idx_all[t].at[pl.ds(nbase * C, C)])

        tabs = (ta_hbm, tb_hbm)
        outs = (oa_hbm, ob_hbm)

        def start_g(j, b):
            for t in range(2):
                pltpu.async_copy(tabs[t].at[idx_all[t].at[pl.ds(j * C, C)]],
                                 bufs[2 * b + t], gsems[2 * b + t])

        for b in range(NB):
            @pl.when(b < nch)
            def _():
                start_g(b, b)

        def outer(j0, _):
            for b in range(NB):
                ch = j0 + b

                @pl.when(ch < nch)
                def _():
                    ws = []
                    for t in range(2):
                        pltpu.make_async_copy(
                            tabs[t].at[idx_all[t].at[pl.ds(0, C)]],
                            bufs[2 * b + t], gsems[2 * b + t]).wait()
                        ws.append(pltpu.async_copy(
                            bufs[2 * b + t],
                            outs[t].at[pl.ds(base + ch * C, C)],
                            wsems[2 * b + t]))

                    @pl.when(ch + NB < nch)
                    def _():
                        for w in ws:
                            w.wait()
                        start_g(ch + NB, b)
            return 0

        nouter = -(-(nbase + 1) // NB)
        lax.fori_loop(0, nouter, lambda j, x: outer(j * NB, x), 0)
        for b in range(NB):
            for t in range(2):
                @pl.when(b < nch)
                def _():
                    pltpu.make_async_copy(bufs[2 * b + t],
                                          outs[t].at[pl.ds(base, C)],
                                          wsems[2 * b + t]).wait()

    return k(tab_a, idx_a, tab_b, idx_b)


def _sc_segment_sum(y, idx, n_out):
    E, D = y.shape
    C = 128
    assert E % C == 0
    ntot = E // C
    nbase = ntot // _NW
    nrem = ntot - nbase * _NW
    rows_per_sub = -(-n_out // (_NS * 8)) * 8
    npad = rows_per_sub * _NS
    zeros = jnp.zeros((rows_per_sub, D), jnp.float32)
    NB = 3
    mesh = plsc.VectorSubcoreMesh(core_axis_name="c", subcore_axis_name="s",
                                  num_cores=_NC, num_subcores=_NS)

    @functools.partial(
        pl.kernel,
        out_type=jax.ShapeDtypeStruct((_NC, npad, D), jnp.float32),
        mesh=mesh,
        scratch_types=[
            [pltpu.VMEM((C,), jnp.int32) for _ in range(NB)],
            [pltpu.VMEM((C, D), jnp.float32) for _ in range(NB)],
            [pltpu.SemaphoreType.DMA for _ in range(NB)],
            [pltpu.SemaphoreType.DMA for _ in range(NB)],
            [pltpu.SemaphoreType.DMA for _ in range(NB)],
            pltpu.VMEM_SHARED((npad, D), jnp.float32),
        ],
    )
    def k(y_hbm, idx_hbm, z_hbm, out_hbm, idx_v, rows_v, semi, semr, sems,
          acc_sh):
        c = lax.axis_index("c")
        s = lax.axis_index("s")
        wid = s * _NC + c
        nch = nbase + (wid < nrem).astype(jnp.int32)
        base = (wid * nbase + jnp.minimum(wid, nrem)) * C
        row0 = s * rows_per_sub
        pltpu.sync_copy(z_hbm, acc_sh.at[pl.ds(row0, rows_per_sub)])
        plsc.subcore_barrier()

        def start_load(j, b):
            bb = base + j * C
            pltpu.async_copy(idx_hbm.at[pl.ds(bb, C)], idx_v[b], semi[b])
            pltpu.async_copy(y_hbm.at[pl.ds(bb, C)], rows_v[b], semr[b])

        for b in range(NB):
            @pl.when(b < nch)
            def _():
                start_load(b, b)

        def outer(j0, _):
            for b in range(NB):
                ch = j0 + b

                @pl.when(ch < nch)
                def _():
                    pltpu.make_async_copy(idx_hbm.at[pl.ds(base, C)],
                                          idx_v[b], semi[b]).wait()
                    pltpu.make_async_copy(y_hbm.at[pl.ds(base, C)],
                                          rows_v[b], semr[b]).wait()
                    w = pltpu.async_copy(rows_v[b], acc_sh.at[idx_v[b]],
                                         sems[b], add=True)

                    @pl.when(ch + NB < nch)
                    def _():
                        w.wait()
                        start_load(ch + NB, b)

            return 0

        nouter = -(-(nbase + 1) // NB)
        lax.fori_loop(0, nouter, lambda j, x: outer(j * NB, x), 0)
        for b in range(NB):
            @pl.when(b < nch)
            def _():
                pltpu.make_async_copy(rows_v[b], acc_sh.at[idx_v[b]],
                                      sems[b]).wait()
        plsc.subcore_barrier()
        pltpu.sync_copy(acc_sh.at[pl.ds(row0, rows_per_sub)],
                        out_hbm.at[c, pl.ds(row0, rows_per_sub)])

    return k(y, idx, zeros)


def _edge_mlp_pallas(Ag, Bg, ex, Wx, b0, W1, b1, maskf):
    E = Ag.shape[0]
    H0 = W1.shape[0]
    H1 = W1.shape[1]
    EB = 512
    assert E % EB == 0

    def body(ag_ref, bg_ref, ex_ref, wx_ref, b0_ref, w1_ref, b1_ref, o_ref):
        def unpack(v):
            lo = jax.lax.bitcast_convert_type(v << 16, jnp.float32)
            hi = jax.lax.bitcast_convert_type(
                v & jnp.int32(-65536), jnp.float32)
            return jnp.concatenate([lo, hi], axis=1)

        pre = (unpack(ag_ref[...]) + unpack(bg_ref[...])
               + ex_ref[...] @ wx_ref[...] + b0_ref[...])
        u = jnp.maximum(pre, 0.0)
        y = jnp.maximum(jnp.dot(u, w1_ref[...], preferred_element_type=jnp.float32)
                        + b1_ref[...], 0.0)
        m = ex_ref[:, 4:5]
        o_ref[...] = y * m

    return pl.pallas_call(
        body,
        grid=(E // EB,),
        in_specs=[
            pl.BlockSpec((EB, H0 // 2), lambda i: (i, 0)),
            pl.BlockSpec((EB, H0 // 2), lambda i: (i, 0)),
            pl.BlockSpec((EB, 8), lambda i: (i, 0)),
            pl.BlockSpec((8, H0), lambda i: (0, 0)),
            pl.BlockSpec((1, H0), lambda i: (0, 0)),
            pl.BlockSpec((H0, H1), lambda i: (0, 0)),
            pl.BlockSpec((1, H1), lambda i: (0, 0)),
        ],
        out_specs=pl.BlockSpec((EB, H1), lambda i: (i, 0)),
        out_shape=jax.ShapeDtypeStruct((E, H1), jnp.float32),
    )(Ag, Bg, ex, Wx, b0.reshape(1, H0), W1, b1.reshape(1, H1))


def _split_msg_weights(p, d_h):
    W0 = p["W0"]
    W_r = W0[:d_h]
    W_s = W0[d_h:2 * d_h]
    W_e = W0[2 * d_h:]
    H0 = W0.shape[1]
    W_d = jnp.zeros((d_h, H0), W0.dtype).at[3:128].set(W_e[0:125])
    Wx = jnp.concatenate([W_e[125:129], jnp.zeros((4, H0), W0.dtype)], axis=0)
    return W_r, W_s, W_d, Wx


def kernel(x, edge_index, i, params):
    n = x.shape[0]
    s = edge_index[0].astype(jnp.int32)
    r = edge_index[1].astype(jnp.int32)
    seg = i.astype(jnp.int32)
    E = s.shape[0]

    xs4 = x[s, :4]
    xr4 = x[r, :4]
    maskf = (xs4[:, 3] <= xr4[:, 3]).astype(jnp.float32)
    d3 = xr4[:, :3] - xs4[:, :3]
    sq = jnp.sum(d3 * d3, axis=1)
    dists = jnp.sqrt(jnp.maximum(sq, 1e-24))
    vects = d3 / dists[:, None]
    ex = jnp.concatenate(
        [dists[:, None], vects, maskf[:, None], jnp.zeros((E, 3), jnp.float32)], axis=1)

    h = x
    for hop, p in enumerate(params["msg"]):
        d_h = h.shape[1]
        W_r, W_s, W_d, Wx = _split_msg_weights(p, d_h)
        if hop == 0:
            A = x @ (W_r + W_d)
            B = x @ (W_s - W_d)
        else:
            A = h @ W_r + x @ W_d
            B = h @ W_s - x @ W_d
        def pack_bf16(M):
            bf = M.astype(jnp.bfloat16).reshape(M.shape[0], M.shape[1] // 2, 2)
            return jax.lax.bitcast_convert_type(bf, jnp.int32)

        evod = lambda v, ax: jnp.concatenate(
            [lax.slice_in_dim(v, 0, None, 2, ax), lax.slice_in_dim(v, 1, None, 2, ax)], ax)
        Ag, Bg = _sc_gather2(pack_bf16(A), r, pack_bf16(B), s)
        y = _edge_mlp_pallas(Ag, Bg, ex, evod(Wx, 1), evod(p["b0"], 0),
                             evod(p["W1"], 0), p["b1"], maskf)
        acc = _sc_segment_sum(y, r, n)
        h = (acc[0, :n] + acc[1, :n])

    pu = params["upd"]
    h = jax.nn.relu(h @ pu["W0"] + pu["b0"])
    h = jax.nn.relu(h @ pu["W1"] + pu["b1"])

    idx_sm = jnp.where(maskf > 0, s, jnp.int32(n))
    hp = jnp.concatenate(
        [h, jnp.ones((n, 1), jnp.float32), jnp.zeros((n, 63), jnp.float32)], axis=1)
    accp = _sc_agg(hp, r, idx_sm, n)
    acc01 = accp[0, :n] + accp[1, :n]
    deg = acc01[:, 64]
    agg = acc01[:, :64] / jnp.maximum(deg, 1.0)[:, None]
    psage = params["sage"]
    out = jnp.concatenate([h, agg], axis=1) @ psage["W"] + psage["b"]
    out = out / jnp.sqrt(jnp.maximum(jnp.sum(out ** 2, axis=-1, keepdims=True), 1e-12))
    h = jax.nn.relu(out)

    p1 = jax.ops.segment_max(h, seg, num_segments=N_GRAPHS)
    cnt = jax.ops.segment_sum(jnp.ones((n,), h.dtype), seg, num_segments=N_GRAPHS)
    psum = jax.ops.segment_sum(h, seg, num_segments=N_GRAPHS)
    p2 = psum / jnp.maximum(cnt, 1.0)[:, None]
    g = jnp.concatenate([p1, p2, psum], axis=1)

    for d in params["dec"]:
        g = g @ d["W"] + d["b"]
        g = jnp.where(g > 0, g, 0.15 * g)
        g = (g - d["mmean"]) / jnp.sqrt(d["mvar"] + 1e-3) * d["gamma"] + d["beta"]

    def dense_stack(layers, v):
        for l in layers:
            v = v @ l["W"] + l["b"]
        return v

    x_loge = dense_stack(params["loge"], g)
    x_ang = dense_stack(params["angles"], g)
    zeniazi = jax.nn.sigmoid(dense_stack(params["angle_scale"], x_ang))
    x_sigs = jnp.abs(dense_stack(params["sigs"], g)) + 1e-5
    xs_out = jnp.stack([x_loge[:, 0], zeniazi[:, 0] * PI, zeniazi[:, 1] * 2.0 * PI], axis=1)
    return jnp.concatenate([xs_out, x_sigs], axis=1)


# speedup vs baseline: 1.0956x; 1.0956x over previous
import functools

import jax
import jax.numpy as jnp
import numpy as np
from jax import lax
from jax.experimental import pallas as pl
from jax.experimental.pallas import tpu as pltpu
from jax.experimental.pallas import tpu_sc as plsc

N_GRAPHS = 16
PI = float(np.pi)

_NC, _NS = 2, 16
_NW = _NC * _NS


def _sc_gather(table, idx):
    E = idx.shape[0]
    n, D = table.shape
    dt = table.dtype
    assert E % _NW == 0
    per_w = E // _NW
    C = 128
    nch = per_w // C
    tail = per_w - nch * C
    assert tail % 8 == 0
    NB = 4
    mesh = plsc.VectorSubcoreMesh(core_axis_name="c", subcore_axis_name="s",
                                  num_cores=_NC, num_subcores=_NS)

    @functools.partial(
        pl.kernel,
        out_type=jax.ShapeDtypeStruct((E, D), dt),
        mesh=mesh,
        scratch_types=[
            pltpu.VMEM((per_w,), jnp.int32),
            [pltpu.VMEM((C, D), dt) for _ in range(NB)],
            [pltpu.SemaphoreType.DMA for _ in range(NB)],
            [pltpu.SemaphoreType.DMA for _ in range(NB)],
            pltpu.VMEM((tail, D), dt) if tail else None,
            pltpu.SemaphoreType.DMA,
        ],
    )
    def k(tab_hbm, idx_hbm, out_hbm, idx_all, bufs, gsems, wsems, buf_t, sem_t):
        c = lax.axis_index("c")
        s = lax.axis_index("s")
        wid = s * _NC + c
        base = wid * per_w
        pltpu.sync_copy(idx_hbm.at[pl.ds(base, per_w)], idx_all)

        def start_g(ch, b):
            pltpu.async_copy(tab_hbm.at[idx_all.at[pl.ds(ch * C, C)]],
                             bufs[b], gsems[b])

        for b in range(NB):
            if b < nch:
                start_g(b, b)

        def outer(j0, _):
            for b in range(NB):
                ch = j0 + b

                @pl.when(ch < nch)
                def _():
                    pltpu.make_async_copy(tab_hbm.at[idx_all.at[pl.ds(ch * C, C)]],
                                          bufs[b], gsems[b]).wait()
                    w = pltpu.async_copy(bufs[b],
                                         out_hbm.at[pl.ds(base + ch * C, C)],
                                         wsems[b])

                    @pl.when(ch + NB < nch)
                    def _():
                        w.wait()
                        start_g(ch + NB, b)
            return 0

        nouter = -(-nch // NB)
        lax.fori_loop(0, nouter, lambda j, x: outer(j * NB, x), 0)
        for b in range(min(NB, nch)):
            pltpu.make_async_copy(bufs[b], out_hbm.at[pl.ds(base, C)],
                                  wsems[b]).wait()
        if tail:
            bt = base + nch * C
            pltpu.async_copy(tab_hbm.at[idx_all.at[pl.ds(nch * C, tail)]],
                             buf_t, sem_t).wait()
            pltpu.sync_copy(buf_t, out_hbm.at[pl.ds(bt, tail)])

    return k(table, idx)


def _sc_agg(hp, idx_g, idx_sc, n_out):
    E = idx_g.shape[0]
    n, D = hp.shape
    C = 128
    assert D == 128 and E % C == 0
    ntot = E // C
    nbase = ntot // _NW
    nrem = ntot - nbase * _NW
    rows_per_sub = -(-(n_out + 8) // (_NS * 8)) * 8
    npad = rows_per_sub * _NS
    z_d = jnp.zeros((rows_per_sub, D), jnp.float32)
    NB = 2
    mesh = plsc.VectorSubcoreMesh(core_axis_name="c", subcore_axis_name="s",
                                  num_cores=_NC, num_subcores=_NS)

    @functools.partial(
        pl.kernel,
        out_type=jax.ShapeDtypeStruct((_NC, npad, D), jnp.float32),
        mesh=mesh,
        scratch_types=[
            pltpu.VMEM(((nbase + 1) * C,), jnp.int32),
            [pltpu.VMEM((C,), jnp.int32) for _ in range(NB)],
            [pltpu.VMEM((C, D), jnp.float32) for _ in range(NB)],
            [pltpu.SemaphoreType.DMA for _ in range(NB)],
            [pltpu.SemaphoreType.DMA for _ in range(NB)],
            [pltpu.SemaphoreType.DMA for _ in range(NB)],
            pltpu.VMEM_SHARED((npad, D), jnp.float32),
        ],
    )
    def k(h_hbm, ig_hbm, is_hbm, zd_hbm, acc_out,
          ig_all, is_v, rows_v, semg, semi, sems, acc_sh):
        c = lax.axis_index("c")
        s = lax.axis_index("s")
        wid = s * _NC + c
        nch = nbase + (wid < nrem).astype(jnp.int32)
        base = (wid * nbase + jnp.minimum(wid, nrem)) * C
        row0 = s * rows_per_sub
        pltpu.sync_copy(zd_hbm, acc_sh.at[pl.ds(row0, rows_per_sub)])
        if nbase:
            pltpu.sync_copy(ig_hbm.at[pl.ds(base, nbase * C)],
                            ig_all.at[pl.ds(0, nbase * C)])

        @pl.when(wid < nrem)
        def _():
            pltpu.sync_copy(ig_hbm.at[pl.ds(base + nbase * C, C)],
                            ig_all.at[pl.ds(nbase * C, C)])

        plsc.subcore_barrier()

        def start_load(j, b):
            pltpu.async_copy(is_hbm.at[pl.ds(base + j * C, C)], is_v[b],
                             semi[b])
            pltpu.async_copy(h_hbm.at[ig_all.at[pl.ds(j * C, C)]], rows_v[b],
                             semg[b])

        for b in range(NB):
            @pl.when(b < nch)
            def _():
                start_load(b, b)

        def outer(j0, _):
            for b in range(NB):
                ch = j0 + b

                @pl.when(ch < nch)
                def _():
                    pltpu.make_async_copy(is_hbm.at[pl.ds(base, C)],
                                          is_v[b], semi[b]).wait()
                    pltpu.make_async_copy(h_hbm.at[ig_all.at[pl.ds(0, C)]],
                                          rows_v[b], semg[b]).wait()
                    w = pltpu.async_copy(rows_v[b], acc_sh.at[is_v[b]],
                                         sems[b], add=True)

                    @pl.when(ch + NB < nch)
                    def _():
                        w.wait()
                        start_load(ch + NB, b)

            return 0

        nouter = -(-(nbase + 1) // NB)
        lax.fori_loop(0, nouter, lambda j, x: outer(j * NB, x), 0)
        for b in range(NB):
            @pl.when(b < nch)
            def _():
                pltpu.make_async_copy(rows_v[b], acc_sh.at[is_v[b]],
                                      sems[b]).wait()
        plsc.subcore_barrier()
        pltpu.sync_copy(acc_sh.at[pl.ds(row0, rows_per_sub)],
                        acc_out.at[c, pl.ds(row0, rows_per_sub)])

    return k(hp, idx_g, idx_sc, z_d)


def _sc_segment_sum(y, idx, n_out):
    E, D = y.shape
    C = 128
    assert E % C == 0
    ntot = E // C
    nbase = ntot // _NW
    nrem = ntot - nbase * _NW
    rows_per_sub = -(-n_out // (_NS * 8)) * 8
    npad = rows_per_sub * _NS
    zeros = jnp.zeros((rows_per_sub, D), jnp.float32)
    NB = 3
    mesh = plsc.VectorSubcoreMesh(core_axis_name="c", subcore_axis_name="s",
                                  num_cores=_NC, num_subcores=_NS)

    @functools.partial(
        pl.kernel,
        out_type=jax.ShapeDtypeStruct((_NC, npad, D), jnp.float32),
        mesh=mesh,
        scratch_types=[
            [pltpu.VMEM((C,), jnp.int32) for _ in range(NB)],
            [pltpu.VMEM((C, D), jnp.float32) for _ in range(NB)],
            [pltpu.SemaphoreType.DMA for _ in range(NB)],
            [pltpu.SemaphoreType.DMA for _ in range(NB)],
            [pltpu.SemaphoreType.DMA for _ in range(NB)],
            pltpu.VMEM_SHARED((npad, D), jnp.float32),
        ],
    )
    def k(y_hbm, idx_hbm, z_hbm, out_hbm, idx_v, rows_v, semi, semr, sems,
          acc_sh):
        c = lax.axis_index("c")
        s = lax.axis_index("s")
        wid = s * _NC + c
        nch = nbase + (wid < nrem).astype(jnp.int32)
        base = (wid * nbase + jnp.minimum(wid, nrem)) * C
        row0 = s * rows_per_sub
        pltpu.sync_copy(z_hbm, acc_sh.at[pl.ds(row0, rows_per_sub)])
        plsc.subcore_barrier()

        def start_load(j, b):
            bb = base + j * C
            pltpu.async_copy(idx_hbm.at[pl.ds(bb, C)], idx_v[b], semi[b])
            pltpu.async_copy(y_hbm.at[pl.ds(bb, C)], rows_v[b], semr[b])

        for b in range(NB):
            @pl.when(b < nch)
            def _():
                start_load(b, b)

        def outer(j0, _):
            for b in range(NB):
                ch = j0 + b

                @pl.when(ch < nch)
                def _():
                    pltpu.make_async_copy(idx_hbm.at[pl.ds(base, C)],
                                          idx_v[b], semi[b]).wait()
                    pltpu.make_async_copy(y_hbm.at[pl.ds(base, C)],
                                          rows_v[b], semr[b]).wait()
                    w = pltpu.async_copy(rows_v[b], acc_sh.at[idx_v[b]],
                                         sems[b], add=True)

                    @pl.when(ch + NB < nch)
                    def _():
                        w.wait()
                        start_load(ch + NB, b)

            return 0

        nouter = -(-(nbase + 1) // NB)
        lax.fori_loop(0, nouter, lambda j, x: outer(j * NB, x), 0)
        for b in range(NB):
            @pl.when(b < nch)
            def _():
                pltpu.make_async_copy(rows_v[b], acc_sh.at[idx_v[b]],
                                      sems[b]).wait()
        plsc.subcore_barrier()
        pltpu.sync_copy(acc_sh.at[pl.ds(row0, rows_per_sub)],
                        out_hbm.at[c, pl.ds(row0, rows_per_sub)])

    return k(y, idx, zeros)


def _edge_mlp_pallas(Ag, Bg, ex, Wx, b0, W1, b1, maskf):
    E = Ag.shape[0]
    H0 = W1.shape[0]
    H1 = W1.shape[1]
    EB = 512
    assert E % EB == 0

    def body(ag_ref, bg_ref, ex_ref, wx_ref, b0_ref, w1_ref, b1_ref, o_ref):
        def unpack(v):
            lo = jax.lax.bitcast_convert_type(v << 16, jnp.float32)
            hi = jax.lax.bitcast_convert_type(
                v & jnp.int32(-65536), jnp.float32)
            return jnp.concatenate([lo, hi], axis=1)

        pre = (unpack(ag_ref[...]) + unpack(bg_ref[...])
               + ex_ref[...] @ wx_ref[...] + b0_ref[...])
        u = jnp.maximum(pre, 0.0)
        y = jnp.maximum(jnp.dot(u, w1_ref[...], preferred_element_type=jnp.float32)
                        + b1_ref[...], 0.0)
        m = ex_ref[:, 4:5]
        o_ref[...] = y * m

    return pl.pallas_call(
        body,
        grid=(E // EB,),
        in_specs=[
            pl.BlockSpec((EB, H0 // 2), lambda i: (i, 0)),
            pl.BlockSpec((EB, H0 // 2), lambda i: (i, 0)),
            pl.BlockSpec((EB, 8), lambda i: (i, 0)),
            pl.BlockSpec((8, H0), lambda i: (0, 0)),
            pl.BlockSpec((1, H0), lambda i: (0, 0)),
            pl.BlockSpec((H0, H1), lambda i: (0, 0)),
            pl.BlockSpec((1, H1), lambda i: (0, 0)),
        ],
        out_specs=pl.BlockSpec((EB, H1), lambda i: (i, 0)),
        out_shape=jax.ShapeDtypeStruct((E, H1), jnp.float32),
    )(Ag, Bg, ex, Wx, b0.reshape(1, H0), W1, b1.reshape(1, H1))


def _upd_pallas(acc0, acc1, W0, b0, W1, b1, n):
    TN = 1000
    nt = n // TN

    def body(a0, a1, w0, b0r, w1, b1r, o):
        hin = a0[...] + a1[...]
        h1 = jnp.maximum(
            jnp.dot(hin, w0[...], preferred_element_type=jnp.float32) + b0r[...], 0.0)
        h2 = jnp.maximum(
            jnp.dot(h1, w1[...], preferred_element_type=jnp.float32) + b1r[...], 0.0)
        o[...] = jnp.concatenate(
            [h2, jnp.ones((TN, 1), jnp.float32), jnp.zeros((TN, 63), jnp.float32)],
            axis=1)

    return pl.pallas_call(
        body,
        grid=(nt,),
        in_specs=[
            pl.BlockSpec((TN, 128), lambda i: (i, 0)),
            pl.BlockSpec((TN, 128), lambda i: (i, 0)),
            pl.BlockSpec((128, 128), lambda i: (0, 0)),
            pl.BlockSpec((1, 128), lambda i: (0, 0)),
            pl.BlockSpec((128, 64), lambda i: (0, 0)),
            pl.BlockSpec((1, 64), lambda i: (0, 0)),
        ],
        out_specs=pl.BlockSpec((TN, 128), lambda i: (i, 0)),
        out_shape=jax.ShapeDtypeStruct((n, 128), jnp.float32),
    )(acc0, acc1, W0, b0.reshape(1, -1), W1, b1.reshape(1, -1))


def _sagepool_pallas(hp, acc0, acc1, W_ext, W_bot, PT, n):
    TN = 1000
    nt = n // TN

    def body(hp_r, a0, a1, wext, wbot, pt_r, p1_o, ps_o):
        i = pl.program_id(0)
        acc = a0[...] + a1[...]
        deg = jnp.maximum(acc[:, 64:65], 1.0)
        agg = acc[:, :64] / deg
        out = (jnp.dot(hp_r[...], wext[...], preferred_element_type=jnp.float32)
               + jnp.dot(agg, wbot[...], preferred_element_type=jnp.float32))
        nrm = jnp.sqrt(jnp.maximum(jnp.sum(out * out, axis=1, keepdims=True), 1e-12))
        h = jnp.maximum(out / nrm, 0.0)

        pt = pt_r[...]
        for g in range(N_GRAPHS):
            col = pt[:, g:g + 1]
            mg = jnp.max(jnp.where(col > 0, h, -3.4e38), axis=0, keepdims=True)
            sg = jnp.sum(h * col, axis=0, keepdims=True)

            @pl.when(i == 0)
            def _():
                p1_o[g:g + 1, :] = mg
                ps_o[g:g + 1, :] = sg

            @pl.when(i > 0)
            def _():
                p1_o[g:g + 1, :] = jnp.maximum(p1_o[g:g + 1, :], mg)
                ps_o[g:g + 1, :] = ps_o[g:g + 1, :] + sg

    return pl.pallas_call(
        body,
        grid=(nt,),
        in_specs=[
            pl.BlockSpec((TN, 128), lambda i: (i, 0)),
            pl.BlockSpec((TN, 128), lambda i: (i, 0)),
            pl.BlockSpec((TN, 128), lambda i: (i, 0)),
            pl.BlockSpec((128, 128), lambda i: (0, 0)),
            pl.BlockSpec((64, 128), lambda i: (0, 0)),
            pl.BlockSpec((TN, N_GRAPHS), lambda i: (i, 0)),
        ],
        out_specs=[
            pl.BlockSpec((N_GRAPHS, 128), lambda i: (0, 0)),
            pl.BlockSpec((N_GRAPHS, 128), lambda i: (0, 0)),
        ],
        out_shape=[
            jax.ShapeDtypeStruct((N_GRAPHS, 128), jnp.float32),
            jax.ShapeDtypeStruct((N_GRAPHS, 128), jnp.float32),
        ],
    )(hp, acc0, acc1, W_ext, W_bot, PT)


def _split_msg_weights(p, d_h):
    W0 = p["W0"]
    W_r = W0[:d_h]
    W_s = W0[d_h:2 * d_h]
    W_e = W0[2 * d_h:]
    H0 = W0.shape[1]
    W_d = jnp.zeros((d_h, H0), W0.dtype).at[3:128].set(W_e[0:125])
    Wx = jnp.concatenate([W_e[125:129], jnp.zeros((4, H0), W0.dtype)], axis=0)
    return W_r, W_s, W_d, Wx


def kernel(x, edge_index, i, params):
    n = x.shape[0]
    s = edge_index[0].astype(jnp.int32)
    r = edge_index[1].astype(jnp.int32)
    seg = i.astype(jnp.int32)
    E = s.shape[0]

    xs4 = x[s, :4]
    xr4 = x[r, :4]
    maskf = (xs4[:, 3] <= xr4[:, 3]).astype(jnp.float32)
    d3 = xr4[:, :3] - xs4[:, :3]
    sq = jnp.sum(d3 * d3, axis=1)
    dists = jnp.sqrt(jnp.maximum(sq, 1e-24))
    vects = d3 / dists[:, None]
    ex = jnp.concatenate(
        [dists[:, None], vects, maskf[:, None], jnp.zeros((E, 3), jnp.float32)], axis=1)

    h = x
    for hop, p in enumerate(params["msg"]):
        d_h = h.shape[1]
        W_r, W_s, W_d, Wx = _split_msg_weights(p, d_h)
        if hop == 0:
            A = x @ (W_r + W_d)
            B = x @ (W_s - W_d)
        else:
            A = h @ W_r + x @ W_d
            B = h @ W_s - x @ W_d
        def pack_bf16(M):
            bf = M.astype(jnp.bfloat16).reshape(M.shape[0], M.shape[1] // 2, 2)
            return jax.lax.bitcast_convert_type(bf, jnp.int32)

        evod = lambda v, ax: jnp.concatenate(
            [lax.slice_in_dim(v, 0, None, 2, ax), lax.slice_in_dim(v, 1, None, 2, ax)], ax)
        Ag = _sc_gather(pack_bf16(A), r)
        Bg = _sc_gather(pack_bf16(B), s)
        y = _edge_mlp_pallas(Ag, Bg, ex, evod(Wx, 1), evod(p["b0"], 0),
                             evod(p["W1"], 0), p["b1"], maskf)
        acc = _sc_segment_sum(y, r, n)
        if hop + 1 < len(params["msg"]):
            h = (acc[0, :n] + acc[1, :n])

    pu = params["upd"]
    hp = _upd_pallas(acc[0, :n], acc[1, :n], pu["W0"], pu["b0"],
                     pu["W1"], pu["b1"], n)

    idx_sm = jnp.where(maskf > 0, s, jnp.int32(n))
    accp = _sc_agg(hp, r, idx_sm, n)

    psage = params["sage"]
    W_ext = jnp.concatenate(
        [psage["W"][:64], psage["b"][None, :], jnp.zeros((63, 128), jnp.float32)],
        axis=0)
    P = (jnp.arange(N_GRAPHS, dtype=jnp.int32)[:, None] == seg[None, :]
         ).astype(jnp.float32)
    p1, psum = _sagepool_pallas(hp, accp[0, :n], accp[1, :n], W_ext,
                                psage["W"][64:], P.T, n)
    cnt = jnp.sum(P, axis=1)
    p2 = psum / jnp.maximum(cnt, 1.0)[:, None]
    g = jnp.concatenate([p1, p2, psum], axis=1)

    for d in params["dec"]:
        g = g @ d["W"] + d["b"]
        g = jnp.where(g > 0, g, 0.15 * g)
        g = (g - d["mmean"]) / jnp.sqrt(d["mvar"] + 1e-3) * d["gamma"] + d["beta"]

    def dense_stack(layers, v):
        for l in layers:
            v = v @ l["W"] + l["b"]
        return v

    x_loge = dense_stack(params["loge"], g)
    x_ang = dense_stack(params["angles"], g)
    zeniazi = jax.nn.sigmoid(dense_stack(params["angle_scale"], x_ang))
    x_sigs = jnp.abs(dense_stack(params["sigs"], g)) + 1e-5
    xs_out = jnp.stack([x_loge[:, 0], zeniazi[:, 0] * PI, zeniazi[:, 1] * 2.0 * PI], axis=1)
    return jnp.concatenate([xs_out, x_sigs], axis=1)
